# baseline (speedup 1.0000x reference)
import functools

import jax
import jax.numpy as jnp
from jax import lax
from jax.experimental import pallas as pl
from jax.experimental.pallas import tpu as pltpu
from jax.experimental.pallas import tpu_sc as plsc

_B, _L, _M, _D, _H = 4, 2048, 16, 768, 512
_SEG_A, _SEG_B = 8, 16
_NTOK = _SEG_A + _SEG_B
_NP = _B * _M
_NW = 32
_PPW = _NP // _NW
_ROWS = _PPW * _NTOK


def _sc_gather_body(idx_hbm, txt_hbm, out_hbm, idx_v, rows_v, mean_v, sem):
    wid = lax.axis_index("s") * 2 + lax.axis_index("c")

    pltpu.sync_copy(idx_hbm.at[pl.ds(wid * _ROWS, _ROWS)], idx_v)
    pltpu.async_copy(txt_hbm.at[idx_v], rows_v, sem).wait()

    def chunk(ci, carry):
        col = ci * 16
        acc0 = rows_v[0, pl.ds(col, 16)]
        for r in range(1, _NTOK):
            acc0 = acc0 + rows_v[r, pl.ds(col, 16)]
        mean_v[0, pl.ds(col, 16)] = acc0 * (1.0 / _NTOK)
        acc1 = rows_v[_NTOK, pl.ds(col, 16)]
        for r in range(_NTOK + 1, 2 * _NTOK):
            acc1 = acc1 + rows_v[r, pl.ds(col, 16)]
        mean_v[1, pl.ds(col, 16)] = acc1 * (1.0 / _NTOK)
        return carry

    lax.fori_loop(0, _D // 16, chunk, 0)

    pltpu.sync_copy(mean_v, out_hbm.at[wid])


_sc_gather = functools.partial(
    pl.kernel,
    mesh=plsc.VectorSubcoreMesh(core_axis_name="c", subcore_axis_name="s",
                                num_cores=2),
    out_type=jax.ShapeDtypeStruct((_NW, 8, _D), jnp.float32),
    scratch_types=[
        pltpu.VMEM((_ROWS,), jnp.int32),
        pltpu.VMEM((_ROWS, _D), jnp.float32),
        pltpu.VMEM((8, _D), jnp.float32),
        pltpu.SemaphoreType.DMA,
    ],
)(_sc_gather_body)


def _tc_body(means_ref, imag_ref, flags_ref, w1_ref, w2_ref, w3_ref,
             out_imag_ref, out_loss_ref):
    m = means_ref[...]
    x = imag_ref[...]
    h = jnp.maximum(jnp.dot(x, w1_ref[...], preferred_element_type=jnp.float32), 0.0)
    h = jnp.maximum(jnp.dot(h, w2_ref[...], preferred_element_type=jnp.float32), 0.0)
    p = jnp.dot(h, w3_ref[...], preferred_element_type=jnp.float32)

    fl = flags_ref[...]
    pn = jnp.sqrt(jnp.sum(p * p, axis=1, keepdims=True))
    mn = jnp.sqrt(jnp.sum(m * m, axis=1, keepdims=True))
    denom = jnp.maximum(pn, 1e-8) * jnp.maximum(mn, 1e-8)
    cos = jnp.sum(p * m, axis=1, keepdims=True) / denom
    loss = jnp.sum(fl * (1.0 - cos), axis=0, keepdims=True)
    cnt = jnp.sum(fl, axis=0, keepdims=True)

    out_imag_ref[...] = jnp.where(fl > 0.0, p, x)
    out_loss_ref[...] = jnp.where(cnt > 0.0, loss / jnp.maximum(cnt, 1.0), 0.0)


def kernel(align_txt_embeds, txt_masks, align_imagine_embeds, imagine_masks,
           sub_instr_segs, noun_phrase_segs, sub_instr_imag_flag, W1, W2, W3):
    segs = jnp.asarray(noun_phrase_segs).astype(jnp.int32)
    s_a = jnp.clip(segs[:, :, 0, 0], 0, _L - _SEG_A)
    s_b = jnp.clip(segs[:, :, 1, 0], 0, _L - _SEG_B)
    base = (jnp.arange(_B, dtype=jnp.int32) * _L)[:, None]
    idx_a = (s_a + base)[..., None] + jnp.arange(_SEG_A, dtype=jnp.int32)
    idx_b = (s_b + base)[..., None] + jnp.arange(_SEG_B, dtype=jnp.int32)
    row_idx = jnp.concatenate([idx_a, idx_b], axis=-1).reshape(-1)
    txt_flat = align_txt_embeds.reshape(_B * _L, _D)

    means_padded = _sc_gather(row_idx, txt_flat)
    means = means_padded[:, :_PPW, :].reshape(_NP, _D)

    flags = jnp.asarray(sub_instr_imag_flag).astype(jnp.float32).reshape(_NP, 1)
    imag = align_imagine_embeds.reshape(_NP, _D)

    out_imag, out_loss = pl.pallas_call(
        _tc_body,
        out_shape=[
            jax.ShapeDtypeStruct((_NP, _D), jnp.float32),
            jax.ShapeDtypeStruct((1, 1), jnp.float32),
        ],
    )(means, imag, flags, W1, W2, W3)

    return (out_loss.reshape(()), out_imag.reshape(_B, _M, _D))


# speedup vs baseline: 1.0285x; 1.0285x over previous
import functools

import jax
import jax.numpy as jnp
from jax import lax
from jax.experimental import pallas as pl
from jax.experimental.pallas import tpu as pltpu
from jax.experimental.pallas import tpu_sc as plsc

_B, _L, _M, _D, _H = 4, 2048, 16, 768, 512
_SEG_A, _SEG_B = 8, 16
_NTOK = _SEG_A + _SEG_B
_NP = _B * _M
_NW = 32
_PPW = _NP // _NW
_ROWS = _PPW * _NTOK


def _sc_gather_body(idx_hbm, txt_hbm, out_hbm, idx_v, rows_v, mean_v,
                    sem0, sem1):
    wid = lax.axis_index("s") * 2 + lax.axis_index("c")

    pltpu.sync_copy(idx_hbm.at[pl.ds(wid * _ROWS, _ROWS)], idx_v)
    copy0 = pltpu.make_async_copy(
        txt_hbm.at[idx_v.at[pl.ds(0, _NTOK)]],
        rows_v.at[pl.ds(0, _NTOK)], sem0)
    copy1 = pltpu.make_async_copy(
        txt_hbm.at[idx_v.at[pl.ds(_NTOK, _NTOK)]],
        rows_v.at[pl.ds(_NTOK, _NTOK)], sem1)
    copy0.start()
    copy1.start()
    copy0.wait()

    def reduce_pair(pair):
        def chunk(ci, carry):
            col = ci * 16
            acc = rows_v[pair * _NTOK, pl.ds(col, 16)]
            for r in range(1, _NTOK):
                acc = acc + rows_v[pair * _NTOK + r, pl.ds(col, 16)]
            mean_v[pair, pl.ds(col, 16)] = acc * (1.0 / _NTOK)
            return carry
        lax.fori_loop(0, _D // 16, chunk, 0)

    reduce_pair(0)
    copy1.wait()
    reduce_pair(1)

    pltpu.sync_copy(mean_v, out_hbm.at[wid])


_sc_gather = functools.partial(
    pl.kernel,
    mesh=plsc.VectorSubcoreMesh(core_axis_name="c", subcore_axis_name="s",
                                num_cores=2),
    out_type=jax.ShapeDtypeStruct((_NW, 8, _D), jnp.float32),
    scratch_types=[
        pltpu.VMEM((_ROWS,), jnp.int32),
        pltpu.VMEM((_ROWS, _D), jnp.float32),
        pltpu.VMEM((8, _D), jnp.float32),
        pltpu.SemaphoreType.DMA,
        pltpu.SemaphoreType.DMA,
    ],
)(_sc_gather_body)


def _mlp_body(imag_ref, w1_ref, w2_ref, w3_ref, proj_ref):
    x = imag_ref[...]
    h = jnp.maximum(jnp.dot(x, w1_ref[...], preferred_element_type=jnp.float32), 0.0)
    h = jnp.maximum(jnp.dot(h, w2_ref[...], preferred_element_type=jnp.float32), 0.0)
    proj_ref[...] = jnp.dot(h, w3_ref[...], preferred_element_type=jnp.float32)


def _combine_body(proj_ref, means_ref, imag_ref, flags_ref,
                  out_imag_ref, out_loss_ref):
    p = proj_ref[...]
    m = means_ref[...]
    x = imag_ref[...]
    fl = flags_ref[...]
    pn = jnp.sqrt(jnp.sum(p * p, axis=1, keepdims=True))
    mn = jnp.sqrt(jnp.sum(m * m, axis=1, keepdims=True))
    denom = jnp.maximum(pn, 1e-8) * jnp.maximum(mn, 1e-8)
    cos = jnp.sum(p * m, axis=1, keepdims=True) / denom
    loss = jnp.sum(fl * (1.0 - cos), axis=0, keepdims=True)
    cnt = jnp.sum(fl, axis=0, keepdims=True)

    out_imag_ref[...] = jnp.where(fl > 0.0, p, x)
    out_loss_ref[...] = jnp.where(cnt > 0.0, loss / jnp.maximum(cnt, 1.0), 0.0)


def kernel(align_txt_embeds, txt_masks, align_imagine_embeds, imagine_masks,
           sub_instr_segs, noun_phrase_segs, sub_instr_imag_flag, W1, W2, W3):
    segs = jnp.asarray(noun_phrase_segs).astype(jnp.int32)
    s_a = jnp.clip(segs[:, :, 0, 0], 0, _L - _SEG_A)
    s_b = jnp.clip(segs[:, :, 1, 0], 0, _L - _SEG_B)
    base = (jnp.arange(_B, dtype=jnp.int32) * _L)[:, None]
    idx_a = (s_a + base)[..., None] + jnp.arange(_SEG_A, dtype=jnp.int32)
    idx_b = (s_b + base)[..., None] + jnp.arange(_SEG_B, dtype=jnp.int32)
    row_idx = jnp.concatenate([idx_a, idx_b], axis=-1).reshape(-1)
    txt_flat = align_txt_embeds.reshape(_B * _L, _D)

    means_padded = _sc_gather(row_idx, txt_flat)
    means = means_padded[:, :_PPW, :].reshape(_NP, _D)

    flags = jnp.asarray(sub_instr_imag_flag).astype(jnp.float32).reshape(_NP, 1)
    imag = align_imagine_embeds.reshape(_NP, _D)

    proj = pl.pallas_call(
        _mlp_body,
        out_shape=jax.ShapeDtypeStruct((_NP, _D), jnp.float32),
    )(imag, W1, W2, W3)

    out_imag, out_loss = pl.pallas_call(
        _combine_body,
        out_shape=[
            jax.ShapeDtypeStruct((_NP, _D), jnp.float32),
            jax.ShapeDtypeStruct((1, 1), jnp.float32),
        ],
    )(proj, means, imag, flags)

    return (out_loss.reshape(()), out_imag.reshape(_B, _M, _D))


# speedup vs baseline: 1.0949x; 1.0646x over previous
import functools

import jax
import jax.numpy as jnp
from jax import lax
from jax.experimental import pallas as pl
from jax.experimental.pallas import tpu as pltpu
from jax.experimental.pallas import tpu_sc as plsc

_B, _L, _M, _D, _H = 4, 2048, 16, 768, 512
_SEG_A, _SEG_B = 8, 16
_NTOK = _SEG_A + _SEG_B
_NP = _B * _M
_NW = 32
_PPW = _NP // _NW
_ROWS = _PPW * _NTOK


def _sc_gather_body(idx_hbm, txt_hbm, out_hbm, idx_v, rows_v, mean_v,
                    stage_v, sem0, sem1):
    c = lax.axis_index("c")
    s = lax.axis_index("s")
    wid = c * 16 + s

    pltpu.sync_copy(idx_hbm.at[pl.ds(wid * _ROWS, _ROWS)], idx_v)
    copy0 = pltpu.make_async_copy(
        txt_hbm.at[idx_v.at[pl.ds(0, _NTOK)]],
        rows_v.at[pl.ds(0, _NTOK)], sem0)
    copy1 = pltpu.make_async_copy(
        txt_hbm.at[idx_v.at[pl.ds(_NTOK, _NTOK)]],
        rows_v.at[pl.ds(_NTOK, _NTOK)], sem1)
    copy0.start()
    copy1.start()
    copy0.wait()

    def reduce_pair(pair):
        def chunk2(ci, carry):
            for half in range(2):
                col = (ci * 2 + half) * 16
                base = pair * _NTOK
                accs = [rows_v[base + a, pl.ds(col, 16)] for a in range(4)]
                for r in range(4, _NTOK):
                    accs[r % 4] = accs[r % 4] + rows_v[base + r, pl.ds(col, 16)]
                tot = (accs[0] + accs[1]) + (accs[2] + accs[3])
                mean_v[pair, pl.ds(col, 16)] = tot * (1.0 / _NTOK)
            return carry
        lax.fori_loop(0, _D // 32, chunk2, 0)

    reduce_pair(0)
    copy1.wait()
    reduce_pair(1)

    pltpu.sync_copy(mean_v.at[pl.ds(0, _PPW)], stage_v.at[pl.ds(s * _PPW, _PPW)])
    plsc.subcore_barrier()

    @pl.when(s < 4)
    def _write_out():
        pltpu.sync_copy(stage_v.at[pl.ds(s * 8, 8)], rows_v.at[pl.ds(0, 8)])
        pltpu.sync_copy(rows_v.at[pl.ds(0, 8)],
                        out_hbm.at[pl.ds(c * 32 + s * 8, 8)])


_sc_gather = functools.partial(
    pl.kernel,
    mesh=plsc.VectorSubcoreMesh(core_axis_name="c", subcore_axis_name="s",
                                num_cores=2),
    out_type=jax.ShapeDtypeStruct((_NP, _D), jnp.float32),
    scratch_types=[
        pltpu.VMEM((_ROWS,), jnp.int32),
        pltpu.VMEM((_ROWS, _D), jnp.float32),
        pltpu.VMEM((_PPW, _D), jnp.float32),
        pltpu.VMEM_SHARED((32, _D), jnp.float32),
        pltpu.SemaphoreType.DMA,
        pltpu.SemaphoreType.DMA,
    ],
)(_sc_gather_body)


def _mlp_body(imag_ref, flags_ref, w1_ref, w2_ref, w3_ref,
              proj_ref, out_imag_ref):
    x = imag_ref[...]
    h = jnp.maximum(jnp.dot(x, w1_ref[...], preferred_element_type=jnp.float32), 0.0)
    h = jnp.maximum(jnp.dot(h, w2_ref[...], preferred_element_type=jnp.float32), 0.0)
    p = jnp.dot(h, w3_ref[...], preferred_element_type=jnp.float32)
    proj_ref[...] = p
    out_imag_ref[...] = jnp.where(flags_ref[...] > 0.0, p, x)


def _combine_body(proj_ref, means_ref, flags_ref, out_loss_ref):
    p = proj_ref[...]
    m = means_ref[...]
    fl = flags_ref[...]
    pn = jnp.sqrt(jnp.sum(p * p, axis=1, keepdims=True))
    mn = jnp.sqrt(jnp.sum(m * m, axis=1, keepdims=True))
    denom = jnp.maximum(pn, 1e-8) * jnp.maximum(mn, 1e-8)
    cos = jnp.sum(p * m, axis=1, keepdims=True) / denom
    loss = jnp.sum(fl * (1.0 - cos), axis=0, keepdims=True)
    cnt = jnp.sum(fl, axis=0, keepdims=True)
    out_loss_ref[...] = jnp.where(cnt > 0.0, loss / jnp.maximum(cnt, 1.0), 0.0)


def kernel(align_txt_embeds, txt_masks, align_imagine_embeds, imagine_masks,
           sub_instr_segs, noun_phrase_segs, sub_instr_imag_flag, W1, W2, W3):
    segs = jnp.asarray(noun_phrase_segs).astype(jnp.int32)
    s_a = jnp.clip(segs[:, :, 0, 0], 0, _L - _SEG_A)
    s_b = jnp.clip(segs[:, :, 1, 0], 0, _L - _SEG_B)
    base = (jnp.arange(_B, dtype=jnp.int32) * _L)[:, None]
    idx_a = (s_a + base)[..., None] + jnp.arange(_SEG_A, dtype=jnp.int32)
    idx_b = (s_b + base)[..., None] + jnp.arange(_SEG_B, dtype=jnp.int32)
    row_idx = jnp.concatenate([idx_a, idx_b], axis=-1).reshape(-1)
    txt_flat = align_txt_embeds.reshape(_B * _L, _D)

    means = _sc_gather(row_idx, txt_flat)

    flags = jnp.asarray(sub_instr_imag_flag).astype(jnp.float32).reshape(_NP, 1)
    imag = align_imagine_embeds.reshape(_NP, _D)

    proj, out_imag = pl.pallas_call(
        _mlp_body,
        out_shape=[
            jax.ShapeDtypeStruct((_NP, _D), jnp.float32),
            jax.ShapeDtypeStruct((_NP, _D), jnp.float32),
        ],
    )(imag, flags, W1, W2, W3)

    out_loss = pl.pallas_call(
        _combine_body,
        out_shape=jax.ShapeDtypeStruct((1, 1), jnp.float32),
    )(proj, means, flags)

    return (out_loss.reshape(()), out_imag.reshape(_B, _M, _D))
